# baseline (speedup 1.0000x reference)
import numpy as np
import jax
import jax.numpy as jnp
from jax import lax
from jax.experimental import pallas as pl
from jax.experimental.pallas import tpu as pltpu
from jax.experimental.pallas import tpu_sc as plsc

_T, _N, _E, _D = 4, 10000, 320000, 128
_NC, _NS, _L = 2, 16, 16
_DW = _D // 2
_EPW = _E // _NS
_K = 96
_IB = 960
_NIB = _EPW // _IB
_IBT = _EPW - _NIB * _IB
_TCH = _IBT // _K
_TAIL = _IBT - _TCH * _K
_RPW = 632
_RPW_LAST = _N - _RPW * (_NS - 1)

_PERM = np.empty(_D, np.int32)
for _j in range(_D // 32):
    for _m in range(16):
        _PERM[_j * 32 + 2 * _m] = _j * 32 + _m
        _PERM[_j * 32 + 2 * _m + 1] = _j * 32 + 16 + _m


def _bcast_lane(vec16, l):
    idx = jnp.full((_L, 1), l, jnp.int32)
    dn = lax.GatherDimensionNumbers(offset_dims=(), collapsed_slice_dims=(0,),
                                    start_index_map=(0,))
    return lax.gather(vec16, idx, dn, (1,),
                      mode=lax.GatherScatterMode.PROMISE_IN_BOUNDS)


def _mm_body(x_ref, w_ref, o_ref):
    o_ref[...] = jnp.dot(x_ref[...], w_ref[...],
                         preferred_element_type=jnp.float32
                         ).astype(jnp.bfloat16)


def _xw_matmul(x_flat, W):
    BN = 2000
    return pl.pallas_call(
        _mm_body,
        grid=(x_flat.shape[0] // BN,),
        in_specs=[
            pl.BlockSpec((BN, _D), lambda i: (i, 0)),
            pl.BlockSpec((_D, _D), lambda i: (0, 0)),
        ],
        out_specs=pl.BlockSpec((BN, _D), lambda i: (i, 0)),
        out_shape=jax.ShapeDtypeStruct((x_flat.shape[0], _D), jnp.bfloat16),
    )(x_flat, W)


def _spmm_body(dst_hbm, src_hbm, val_hbm, xw_hbm, out_hbm,
               acc, src_b, dst_b, val_b, dst_v0, dst_v1, dst_tail_v,
               rows0, rows1, stg0, stg1,
               sem_g0, sem_g1, sem_s0, sem_s1, sem_i):
    c = lax.axis_index("c")
    s = lax.axis_index("s")

    def _copy_dst(off_e, dvr):
        for j in range(_K // _L):
            dvr[pl.ds(j * _L, _L)] = dst_b[pl.ds(off_e + j * _L, _L)]

    def _scale(rows, stg, off_e, ngroups=_K // _L):
        def _sc(g, c2):
            val16 = val_b[pl.ds(off_e + g * _L, _L)]
            for l in range(_L):
                bc = _bcast_lane(val16, l)
                k = g * _L + l
                for j in range(_DW // _L):
                    w = rows[k, pl.ds(j * _L, _L)]
                    lo = lax.bitcast_convert_type(w << 16, jnp.float32)
                    hi = lax.bitcast_convert_type(
                        w & jnp.int32(-65536), jnp.float32)
                    stg[k, pl.ds(j * 2 * _L, _L)] = lo * bc
                    stg[k, pl.ds(j * 2 * _L + _L, _L)] = hi * bc
            return c2
        lax.fori_loop(0, ngroups, _sc, 0)

    def _issue_gather(off_e, rows, sem, n=_K):
        return pltpu.async_copy(xw_hbm.at[src_b.at[pl.ds(off_e, n)]],
                                rows.at[pl.ds(0, n)] if n != _K else rows,
                                sem)

    def _wait_gather(rows, sem):
        pltpu.make_async_copy(xw_hbm.at[pl.ds(0, _K)], rows, sem).wait()

    def _issue_scatter(stg, dvr, sem):
        pltpu.async_copy(stg, acc.at[dvr], sem, add=True)

    def _wait_scatter(stg, dvr, sem):
        pltpu.make_async_copy(stg, acc.at[dvr], sem).wait()

    def _load_idx_block(eb, n, t):
        d1 = pltpu.async_copy(src_hbm.at[pl.ds(eb, n)],
                              src_b.at[pl.ds(0, n)], sem_i)
        d2 = pltpu.async_copy(dst_hbm.at[pl.ds(eb, n)],
                              dst_b.at[pl.ds(0, n)], sem_i)
        d3 = pltpu.async_copy(val_hbm.at[pl.ds(eb, n)],
                              val_b.at[pl.ds(0, n)], sem_i)
        d1.wait(); d2.wait(); d3.wait()

        def _gl(i, carry):
            b = i * _L
            src_b[pl.ds(b, _L)] = src_b[pl.ds(b, _L)] + t * _N
            return carry
        lax.fori_loop(0, n // _L, _gl, 0)

    def _run_block(nch, first):
        npair = nch // 2
        _issue_gather(0, rows0, sem_g0)

        def _pair(q2, carry):
            off0 = q2 * 2 * _K
            off1 = off0 + _K
            skip = jnp.logical_and(first, q2 == 0)

            _wait_gather(rows0, sem_g0)
            _issue_gather(off1, rows1, sem_g1)

            @pl.when(jnp.logical_not(skip))
            def _():
                _wait_scatter(stg0, dst_v0, sem_s0)
            _scale(rows0, stg0, off0)
            _copy_dst(off0, dst_v0)
            _issue_scatter(stg0, dst_v0, sem_s0)

            _wait_gather(rows1, sem_g1)

            @pl.when(q2 < npair - 1)
            def _():
                _issue_gather(off1 + _K, rows0, sem_g0)

            @pl.when(jnp.logical_not(skip))
            def _():
                _wait_scatter(stg1, dst_v1, sem_s1)
            _scale(rows1, stg1, off1)
            _copy_dst(off1, dst_v1)
            _issue_scatter(stg1, dst_v1, sem_s1)
            return carry
        lax.fori_loop(0, npair, _pair, 0)

    for tt in range(_T // _NC):
        t = c * (_T // _NC) + tt

        def _zf(k, carry):
            for j in range(_D // _L):
                stg0[k, pl.ds(j * _L, _L)] = jnp.zeros((_L,), jnp.float32)
            return carry
        lax.fori_loop(0, _K, _zf, 0)
        r0 = s * _RPW

        def _zero_stripe(rows):
            for q in range(rows // _K):
                pltpu.sync_copy(stg0, acc.at[pl.ds(r0 + q * _K, _K)])
            rem = rows - (rows // _K) * _K
            if rem:
                pltpu.sync_copy(stg0.at[pl.ds(0, rem)],
                                acc.at[pl.ds(r0 + (rows // _K) * _K, rem)])

        @pl.when(s < _NS - 1)
        def _():
            _zero_stripe(_RPW)

        @pl.when(s == _NS - 1)
        def _():
            _zero_stripe(_RPW_LAST)

        plsc.subcore_barrier()

        ebase = t * _E + s * _EPW

        def _block(ib, carry):
            _load_idx_block(ebase + ib * _IB, _IB, t)
            _run_block(_IB // _K, ib == 0)
            return carry
        lax.fori_loop(0, _NIB, _block, 0)

        _load_idx_block(ebase + _NIB * _IB, _IBT, t)
        _run_block(_TCH, jnp.bool_(False))

        _wait_scatter(stg0, dst_v0, sem_s0)
        _wait_scatter(stg1, dst_v1, sem_s1)

        off_e = _TCH * _K
        for j in range(_TAIL // _L):
            dst_tail_v[pl.ds(j * _L, _L)] = dst_b[pl.ds(off_e + j * _L, _L)]
        _issue_gather(off_e, rows0, sem_g0, n=_TAIL).wait()
        _scale(rows0, stg0, off_e, ngroups=_TAIL // _L)
        pltpu.sync_copy(stg0.at[pl.ds(0, _TAIL)], acc.at[dst_tail_v],
                        add=True)

        plsc.subcore_barrier()

        obase = t * _N + r0

        @pl.when(s < _NS - 1)
        def _():
            pltpu.sync_copy(acc.at[pl.ds(r0, _RPW)],
                            out_hbm.at[pl.ds(obase, _RPW)])

        @pl.when(s == _NS - 1)
        def _():
            pltpu.sync_copy(acc.at[pl.ds(r0, _RPW_LAST)],
                            out_hbm.at[pl.ds(obase, _RPW_LAST)])


def kernel(adj_indices, adj_values, input, M, W):
    dst = adj_indices[:, 0, :].reshape(-1)
    src = adj_indices[:, 1, :].reshape(-1)
    val = adj_values.reshape(-1)
    x_flat = input.reshape(_T * _N, _D)
    xw_bf = _xw_matmul(x_flat, W[:, _PERM])
    xw_i32 = lax.bitcast_convert_type(
        xw_bf.reshape(_T * _N, _DW, 2), jnp.int32)

    mesh = plsc.VectorSubcoreMesh(core_axis_name="c", subcore_axis_name="s")
    spmm = pl.kernel(
        _spmm_body,
        out_type=jax.ShapeDtypeStruct((_T * _N, _D), jnp.float32),
        compiler_params=pltpu.CompilerParams(use_tc_tiling_on_sc=False),
        mesh=mesh,
        scratch_types=[
            pltpu.VMEM_SHARED((_N, _D), jnp.float32),
            pltpu.VMEM((_IB,), jnp.int32),
            pltpu.VMEM((_IB,), jnp.int32),
            pltpu.VMEM((_IB,), jnp.float32),
            pltpu.VMEM((_K,), jnp.int32),
            pltpu.VMEM((_K,), jnp.int32),
            pltpu.VMEM((_TAIL,), jnp.int32),
            pltpu.VMEM((_K, _DW), jnp.int32),
            pltpu.VMEM((_K, _DW), jnp.int32),
            pltpu.VMEM((_K, _D), jnp.float32),
            pltpu.VMEM((_K, _D), jnp.float32),
            pltpu.SemaphoreType.DMA,
            pltpu.SemaphoreType.DMA,
            pltpu.SemaphoreType.DMA,
            pltpu.SemaphoreType.DMA,
            pltpu.SemaphoreType.DMA,
        ],
    )
    out = spmm(dst, src, val, xw_i32)
    return out.reshape(_T, _N, _D)


# speedup vs baseline: 1.9625x; 1.9625x over previous
import jax
import jax.numpy as jnp
from jax import lax
from jax.experimental import pallas as pl
from jax.experimental.pallas import tpu as pltpu
from jax.experimental.pallas import tpu_sc as plsc

_T, _N, _E, _D = 4, 10000, 320000, 128
_NC, _NS, _L = 2, 16, 16
_EPW = _E // _NS
_K = 128
_IB = 1280
_NIB = _EPW // _IB
_IBT = _EPW - _NIB * _IB
_TCH = _IBT // _K
_TAIL = _IBT - _TCH * _K
_RPW = 632
_RPW_LAST = _N - _RPW * (_NS - 1)


def _bcast_lane(vec16, l):
    idx = jnp.full((_L, 1), l, jnp.int32)
    dn = lax.GatherDimensionNumbers(offset_dims=(), collapsed_slice_dims=(0,),
                                    start_index_map=(0,))
    return lax.gather(vec16, idx, dn, (1,),
                      mode=lax.GatherScatterMode.PROMISE_IN_BOUNDS)


def _mm_body(x_ref, w_ref, o_ref):
    o_ref[...] = jnp.dot(x_ref[...], w_ref[...],
                         preferred_element_type=jnp.float32)


def _xw_matmul(x_flat, W):
    BN = 2000
    return pl.pallas_call(
        _mm_body,
        grid=(x_flat.shape[0] // BN,),
        in_specs=[
            pl.BlockSpec((BN, _D), lambda i: (i, 0)),
            pl.BlockSpec((_D, _D), lambda i: (0, 0)),
        ],
        out_specs=pl.BlockSpec((BN, _D), lambda i: (i, 0)),
        out_shape=jax.ShapeDtypeStruct((x_flat.shape[0], _D), jnp.float32),
    )(x_flat, W)


def _spmm_body(dst_hbm, src_hbm, val_hbm, xw_hbm, out_hbm,
               acc, src_b, dst_b, val_b, dst_v0, dst_v1, dst_tail_v,
               rows0, rows1, sem_g0, sem_g1, sem_s0, sem_s1, sem_i):
    c = lax.axis_index("c")
    s = lax.axis_index("s")

    def _copy_dst(off_e, dvr):
        for j in range(_K // _L):
            dvr[pl.ds(j * _L, _L)] = dst_b[pl.ds(off_e + j * _L, _L)]

    def _scale(rows, off_e):
        def _sc(g, c2):
            val16 = val_b[pl.ds(off_e + g * _L, _L)]
            for l in range(_L):
                bc = _bcast_lane(val16, l)
                k = g * _L + l
                for j in range(_D // _L):
                    rows[k, pl.ds(j * _L, _L)] = rows[k, pl.ds(j * _L, _L)] * bc
            return c2
        lax.fori_loop(0, _K // _L, _sc, 0)

    def _issue_gather(off_e, rows, sem):
        pltpu.async_copy(xw_hbm.at[src_b.at[pl.ds(off_e, _K)]], rows, sem)

    def _wait_gather(rows, sem):
        pltpu.make_async_copy(xw_hbm.at[pl.ds(0, _K)], rows, sem).wait()

    def _issue_scatter(rows, dvr, sem):
        pltpu.async_copy(rows, acc.at[dvr], sem, add=True)

    def _wait_scatter(rows, dvr, sem):
        pltpu.make_async_copy(rows, acc.at[dvr], sem).wait()

    def _load_idx_block(eb, n, t):
        d1 = pltpu.async_copy(src_hbm.at[pl.ds(eb, n)],
                              src_b.at[pl.ds(0, n)], sem_i)
        d2 = pltpu.async_copy(dst_hbm.at[pl.ds(eb, n)],
                              dst_b.at[pl.ds(0, n)], sem_i)
        d3 = pltpu.async_copy(val_hbm.at[pl.ds(eb, n)],
                              val_b.at[pl.ds(0, n)], sem_i)
        d1.wait(); d2.wait(); d3.wait()

        def _gl(i, carry):
            b = i * _L
            src_b[pl.ds(b, _L)] = src_b[pl.ds(b, _L)] + t * _N
            return carry
        lax.fori_loop(0, n // _L, _gl, 0)

    def _run_block(nch, first):
        npair = nch // 2

        _copy_dst(0, dst_v0)
        _issue_gather(0, rows0, sem_g0)

        def _pair(q2, carry):
            off0 = q2 * 2 * _K
            off1 = off0 + _K
            _wait_gather(rows0, sem_g0)

            @pl.when(jnp.logical_not(jnp.logical_and(first, q2 == 0)))
            def _():
                _wait_scatter(rows1, dst_v1, sem_s1)
            _copy_dst(off1, dst_v1)
            _issue_gather(off1, rows1, sem_g1)

            _scale(rows0, off0)
            _issue_scatter(rows0, dst_v0, sem_s0)

            _wait_gather(rows1, sem_g1)
            _scale(rows1, off1)
            _wait_scatter(rows0, dst_v0, sem_s0)

            @pl.when(q2 < npair - 1)
            def _():
                _copy_dst(off1 + _K, dst_v0)
                _issue_gather(off1 + _K, rows0, sem_g0)
            _issue_scatter(rows1, dst_v1, sem_s1)
            return carry
        lax.fori_loop(0, npair, _pair, 0)

    for tt in range(_T // _NC):
        t = c * (_T // _NC) + tt

        def _zf(k, carry):
            for j in range(_D // _L):
                rows0[k, pl.ds(j * _L, _L)] = jnp.zeros((_L,), jnp.float32)
            return carry
        lax.fori_loop(0, _K, _zf, 0)
        r0 = s * _RPW

        def _zero_stripe(rows):
            for q in range(rows // _K):
                pltpu.sync_copy(rows0, acc.at[pl.ds(r0 + q * _K, _K)])
            rem = rows - (rows // _K) * _K
            if rem:
                pltpu.sync_copy(rows0.at[pl.ds(0, rem)],
                                acc.at[pl.ds(r0 + (rows // _K) * _K, rem)])

        @pl.when(s < _NS - 1)
        def _():
            _zero_stripe(_RPW)

        @pl.when(s == _NS - 1)
        def _():
            _zero_stripe(_RPW_LAST)

        plsc.subcore_barrier()

        ebase = t * _E + s * _EPW

        def _block(ib, carry):
            _load_idx_block(ebase + ib * _IB, _IB, t)
            _run_block(_IB // _K, ib == 0)
            return carry
        lax.fori_loop(0, _NIB, _block, 0)

        _load_idx_block(ebase + _NIB * _IB, _IBT, t)
        _run_block(_TCH, jnp.bool_(False))

        off_e = _TCH * _K
        for j in range(_TAIL // _L):
            dst_tail_v[pl.ds(j * _L, _L)] = dst_b[pl.ds(off_e + j * _L, _L)]
        pltpu.async_copy(xw_hbm.at[src_b.at[pl.ds(off_e, _TAIL)]],
                         rows0.at[pl.ds(0, _TAIL)], sem_g0).wait()

        def _scale_tail(g, c2):
            val16 = val_b[pl.ds(off_e + g * _L, _L)]
            for l in range(_L):
                bc = _bcast_lane(val16, l)
                k = g * _L + l
                for j in range(_D // _L):
                    rows0[k, pl.ds(j * _L, _L)] = (
                        rows0[k, pl.ds(j * _L, _L)] * bc)
            return c2
        lax.fori_loop(0, _TAIL // _L, _scale_tail, 0)

        pltpu.sync_copy(rows0.at[pl.ds(0, _TAIL)], acc.at[dst_tail_v],
                        add=True)
        _wait_scatter(rows1, dst_v1, sem_s1)

        plsc.subcore_barrier()

        obase = t * _N + r0

        @pl.when(s < _NS - 1)
        def _():
            pltpu.sync_copy(acc.at[pl.ds(r0, _RPW)],
                            out_hbm.at[pl.ds(obase, _RPW)])

        @pl.when(s == _NS - 1)
        def _():
            pltpu.sync_copy(acc.at[pl.ds(r0, _RPW_LAST)],
                            out_hbm.at[pl.ds(obase, _RPW_LAST)])


def kernel(adj_indices, adj_values, input, M, W):
    dst = adj_indices[:, 0, :].reshape(-1)
    src = adj_indices[:, 1, :].reshape(-1)
    val = adj_values.reshape(-1)
    x_flat = input.reshape(_T * _N, _D)
    xw = _xw_matmul(x_flat, W)

    mesh = plsc.VectorSubcoreMesh(core_axis_name="c", subcore_axis_name="s")
    spmm = pl.kernel(
        _spmm_body,
        out_type=jax.ShapeDtypeStruct((_T * _N, _D), jnp.float32),
        compiler_params=pltpu.CompilerParams(use_tc_tiling_on_sc=False),
        mesh=mesh,
        scratch_types=[
            pltpu.VMEM_SHARED((_N, _D), jnp.float32),
            pltpu.VMEM((_IB,), jnp.int32),
            pltpu.VMEM((_IB,), jnp.int32),
            pltpu.VMEM((_IB,), jnp.float32),
            pltpu.VMEM((_K,), jnp.int32),
            pltpu.VMEM((_K,), jnp.int32),
            pltpu.VMEM((_TAIL,), jnp.int32),
            pltpu.VMEM((_K, _D), jnp.float32),
            pltpu.VMEM((_K, _D), jnp.float32),
            pltpu.SemaphoreType.DMA,
            pltpu.SemaphoreType.DMA,
            pltpu.SemaphoreType.DMA,
            pltpu.SemaphoreType.DMA,
            pltpu.SemaphoreType.DMA,
        ],
    )
    out = spmm(dst, src, val, xw)
    return out.reshape(_T, _N, _D)


# speedup vs baseline: 2.3666x; 1.2059x over previous
import jax
import jax.numpy as jnp
from jax import lax
from jax.experimental import pallas as pl
from jax.experimental.pallas import tpu as pltpu
from jax.experimental.pallas import tpu_sc as plsc

_T, _N, _E, _D = 4, 10000, 320000, 128
_NC, _NS, _L = 2, 16, 16
_EPW = _E // _NS
_K = 32
_NBG = 4
_CPB = 24
_IB = _CPB * _K
_NIB = _EPW // _IB
_TAIL = _EPW - _NIB * _IB
_RPW = 632
_RPW_LAST = _N - _RPW * (_NS - 1)


def _bcast_lane(vec16, l):
    idx = jnp.full((_L, 1), l, jnp.int32)
    dn = lax.GatherDimensionNumbers(offset_dims=(), collapsed_slice_dims=(0,),
                                    start_index_map=(0,))
    return lax.gather(vec16, idx, dn, (1,),
                      mode=lax.GatherScatterMode.PROMISE_IN_BOUNDS)


def _mm_body(x_ref, w_ref, o_ref):
    o_ref[...] = jnp.dot(x_ref[...], w_ref[...],
                         preferred_element_type=jnp.float32)


def _xw_matmul(x_flat, W):
    BN = 2000
    return pl.pallas_call(
        _mm_body,
        grid=(x_flat.shape[0] // BN,),
        in_specs=[
            pl.BlockSpec((BN, _D), lambda i: (i, 0)),
            pl.BlockSpec((_D, _D), lambda i: (0, 0)),
        ],
        out_specs=pl.BlockSpec((BN, _D), lambda i: (i, 0)),
        out_shape=jax.ShapeDtypeStruct((x_flat.shape[0], _D), jnp.float32),
    )(x_flat, W)


def _spmm_body(dst_hbm, src_hbm, val_hbm, xw_hbm, out_hbm,
               acc, src_b, dst_b, val_b, dst_v0, dst_v1, dst_tail_v,
               g0, g1, g2, g3, stg0, stg1,
               sg0, sg1, sg2, sg3, ss0, ss1, sem_i):
    c = lax.axis_index("c")
    s = lax.axis_index("s")
    gbufs = [g0, g1, g2, g3]
    gsems = [sg0, sg1, sg2, sg3]
    stgs = [stg0, stg1]
    ssems = [ss0, ss1]
    dvs = [dst_v0, dst_v1]

    def _copy_dst(off_e, dvr):
        for j in range(_K // _L):
            dvr[pl.ds(j * _L, _L)] = dst_b[pl.ds(off_e + j * _L, _L)]

    def _scale(rows, stg, off_e, ngroups=_K // _L):
        def _sc(gg, c2):
            val16 = val_b[pl.ds(off_e + gg * _L, _L)]
            for l in range(_L):
                bc = _bcast_lane(val16, l)
                k = gg * _L + l
                for j in range(_D // _L):
                    stg[k, pl.ds(j * _L, _L)] = (
                        rows[k, pl.ds(j * _L, _L)] * bc)
            return c2
        lax.fori_loop(0, ngroups, _sc, 0)

    def _issue_gather(off_e, rows, sem):
        pltpu.async_copy(xw_hbm.at[src_b.at[pl.ds(off_e, _K)]], rows, sem)

    def _wait_gather(rows, sem):
        pltpu.make_async_copy(xw_hbm.at[pl.ds(0, _K)], rows, sem).wait()

    def _issue_scatter(stg, dvr, sem):
        pltpu.async_copy(stg, acc.at[dvr], sem, add=True)

    def _wait_scatter(stg, dvr, sem):
        pltpu.make_async_copy(stg, acc.at[dvr], sem).wait()

    def _load_idx_block(eb, n, t):
        d1 = pltpu.async_copy(src_hbm.at[pl.ds(eb, n)],
                              src_b.at[pl.ds(0, n)], sem_i)
        d2 = pltpu.async_copy(dst_hbm.at[pl.ds(eb, n)],
                              dst_b.at[pl.ds(0, n)], sem_i)
        d3 = pltpu.async_copy(val_hbm.at[pl.ds(eb, n)],
                              val_b.at[pl.ds(0, n)], sem_i)
        d1.wait(); d2.wait(); d3.wait()

        def _gl(i, carry):
            b = i * _L
            src_b[pl.ds(b, _L)] = src_b[pl.ds(b, _L)] + t * _N
            return carry
        lax.fori_loop(0, n // _L, _gl, 0)

    def _run_block(nch, first):
        ngrp = nch // _NBG
        for b in range(_NBG):
            _issue_gather(b * _K, gbufs[b], gsems[b])

        def _grp(g, carry):
            qb0 = g * _NBG
            for b in range(_NBG):
                sb = b % 2
                qb = qb0 + b
                _wait_gather(gbufs[b], gsems[b])
                if b < 2:
                    @pl.when(jnp.logical_not(
                        jnp.logical_and(first, g == 0)))
                    def _():
                        _wait_scatter(stgs[sb], dvs[sb], ssems[sb])
                else:
                    _wait_scatter(stgs[sb], dvs[sb], ssems[sb])
                _scale(gbufs[b], stgs[sb], qb * _K)

                @pl.when(g < ngrp - 1)
                def _():
                    _issue_gather((qb + _NBG) * _K, gbufs[b], gsems[b])
                _copy_dst(qb * _K, dvs[sb])
                _issue_scatter(stgs[sb], dvs[sb], ssems[sb])
            return carry
        lax.fori_loop(0, ngrp, _grp, 0)

    for tt in range(_T // _NC):
        t = c * (_T // _NC) + tt

        def _zf(k, carry):
            for j in range(_D // _L):
                stg0[k, pl.ds(j * _L, _L)] = jnp.zeros((_L,), jnp.float32)
            return carry
        lax.fori_loop(0, _K, _zf, 0)
        r0 = s * _RPW

        def _zero_stripe(rows):
            for q in range(rows // _K):
                pltpu.sync_copy(stg0, acc.at[pl.ds(r0 + q * _K, _K)])
            rem = rows - (rows // _K) * _K
            if rem:
                pltpu.sync_copy(stg0.at[pl.ds(0, rem)],
                                acc.at[pl.ds(r0 + (rows // _K) * _K, rem)])

        @pl.when(s < _NS - 1)
        def _():
            _zero_stripe(_RPW)

        @pl.when(s == _NS - 1)
        def _():
            _zero_stripe(_RPW_LAST)

        plsc.subcore_barrier()

        ebase = t * _E + s * _EPW

        def _block(ib, carry):
            _load_idx_block(ebase + ib * _IB, _IB, t)
            _run_block(_CPB, ib == 0)
            return carry
        lax.fori_loop(0, _NIB, _block, 0)

        _wait_scatter(stg0, dst_v0, ss0)
        _wait_scatter(stg1, dst_v1, ss1)

        _load_idx_block(ebase + _NIB * _IB, _TAIL, t)
        off_e = 0
        for j in range(_TAIL // _L):
            dst_tail_v[pl.ds(j * _L, _L)] = dst_b[pl.ds(off_e + j * _L, _L)]
        pltpu.async_copy(xw_hbm.at[src_b.at[pl.ds(off_e, _TAIL)]],
                         g0.at[pl.ds(0, _TAIL)], sg0).wait()
        _scale(g0, stg0, off_e, ngroups=_TAIL // _L)
        pltpu.sync_copy(stg0.at[pl.ds(0, _TAIL)], acc.at[dst_tail_v],
                        add=True)

        plsc.subcore_barrier()

        obase = t * _N + r0

        @pl.when(s < _NS - 1)
        def _():
            pltpu.sync_copy(acc.at[pl.ds(r0, _RPW)],
                            out_hbm.at[pl.ds(obase, _RPW)])

        @pl.when(s == _NS - 1)
        def _():
            pltpu.sync_copy(acc.at[pl.ds(r0, _RPW_LAST)],
                            out_hbm.at[pl.ds(obase, _RPW_LAST)])


def kernel(adj_indices, adj_values, input, M, W):
    dst = adj_indices[:, 0, :].reshape(-1)
    src = adj_indices[:, 1, :].reshape(-1)
    val = adj_values.reshape(-1)
    x_flat = input.reshape(_T * _N, _D)
    xw = _xw_matmul(x_flat, W)

    mesh = plsc.VectorSubcoreMesh(core_axis_name="c", subcore_axis_name="s")
    spmm = pl.kernel(
        _spmm_body,
        out_type=jax.ShapeDtypeStruct((_T * _N, _D), jnp.float32),
        compiler_params=pltpu.CompilerParams(use_tc_tiling_on_sc=False),
        mesh=mesh,
        scratch_types=[
            pltpu.VMEM_SHARED((_N, _D), jnp.float32),
            pltpu.VMEM((_IB,), jnp.int32),
            pltpu.VMEM((_IB,), jnp.int32),
            pltpu.VMEM((_IB,), jnp.float32),
            pltpu.VMEM((_K,), jnp.int32),
            pltpu.VMEM((_K,), jnp.int32),
            pltpu.VMEM((_TAIL,), jnp.int32),
            pltpu.VMEM((_K, _D), jnp.float32),
            pltpu.VMEM((_K, _D), jnp.float32),
            pltpu.VMEM((_K, _D), jnp.float32),
            pltpu.VMEM((_K, _D), jnp.float32),
            pltpu.VMEM((_K, _D), jnp.float32),
            pltpu.VMEM((_K, _D), jnp.float32),
            pltpu.SemaphoreType.DMA,
            pltpu.SemaphoreType.DMA,
            pltpu.SemaphoreType.DMA,
            pltpu.SemaphoreType.DMA,
            pltpu.SemaphoreType.DMA,
            pltpu.SemaphoreType.DMA,
            pltpu.SemaphoreType.DMA,
        ],
    )
    out = spmm(dst, src, val, xw)
    return out.reshape(_T, _N, _D)


# speedup vs baseline: 2.5098x; 1.0605x over previous
import jax
import jax.numpy as jnp
from jax import lax
from jax.experimental import pallas as pl
from jax.experimental.pallas import tpu as pltpu
from jax.experimental.pallas import tpu_sc as plsc

_T, _N, _E, _D = 4, 10000, 320000, 128
_NC, _NS, _L = 2, 16, 16
_EPW = _E // _NS
_K = 32
_NBG = 4
_CPB = 48
_IB = _CPB * _K
_NIB = _EPW // _IB
_TAIL = _EPW - _NIB * _IB
_RPW = 632
_RPW_LAST = _N - _RPW * (_NS - 1)


def _bcast_lane(vec16, l):
    idx = jnp.full((_L, 1), l, jnp.int32)
    dn = lax.GatherDimensionNumbers(offset_dims=(), collapsed_slice_dims=(0,),
                                    start_index_map=(0,))
    return lax.gather(vec16, idx, dn, (1,),
                      mode=lax.GatherScatterMode.PROMISE_IN_BOUNDS)


def _mm_body(x_ref, w_ref, o_ref):
    o_ref[...] = jnp.dot(x_ref[...], w_ref[...],
                         preferred_element_type=jnp.float32)


def _xw_matmul(x_flat, W):
    BN = 2000
    return pl.pallas_call(
        _mm_body,
        grid=(x_flat.shape[0] // BN,),
        in_specs=[
            pl.BlockSpec((BN, _D), lambda i: (i, 0)),
            pl.BlockSpec((_D, _D), lambda i: (0, 0)),
        ],
        out_specs=pl.BlockSpec((BN, _D), lambda i: (i, 0)),
        out_shape=jax.ShapeDtypeStruct((x_flat.shape[0], _D), jnp.float32),
    )(x_flat, W)


def _spmm_body(dst_hbm, src_hbm, val_hbm, xw_hbm, out_hbm,
               acc, src_b, dst_b, val_b, dst_v0, dst_v1, dst_tail_v,
               g0, g1, g2, g3, stg0, stg1,
               sg0, sg1, sg2, sg3, ss0, ss1, sem_i):
    c = lax.axis_index("c")
    s = lax.axis_index("s")
    gbufs = [g0, g1, g2, g3]
    gsems = [sg0, sg1, sg2, sg3]
    stgs = [stg0, stg1]
    ssems = [ss0, ss1]
    dvs = [dst_v0, dst_v1]

    def _copy_dst(off_e, dvr):
        for j in range(_K // _L):
            dvr[pl.ds(j * _L, _L)] = dst_b[pl.ds(off_e + j * _L, _L)]

    def _scale(rows, stg, off_e, ngroups=_K // _L):
        def _sc(gg, c2):
            val16 = val_b[pl.ds(off_e + gg * _L, _L)]
            for l in range(_L):
                bc = _bcast_lane(val16, l)
                k = gg * _L + l
                for j in range(_D // _L):
                    stg[k, pl.ds(j * _L, _L)] = (
                        rows[k, pl.ds(j * _L, _L)] * bc)
            return c2
        lax.fori_loop(0, ngroups, _sc, 0)

    def _issue_gather(off_e, rows, sem):
        pltpu.async_copy(xw_hbm.at[src_b.at[pl.ds(off_e, _K)]], rows, sem)

    def _wait_gather(rows, sem):
        pltpu.make_async_copy(xw_hbm.at[pl.ds(0, _K)], rows, sem).wait()

    def _issue_scatter(stg, dvr, sem):
        pltpu.async_copy(stg, acc.at[dvr], sem, add=True)

    def _wait_scatter(stg, dvr, sem):
        pltpu.make_async_copy(stg, acc.at[dvr], sem).wait()

    def _load_idx_block(eb, n, t):
        d1 = pltpu.async_copy(src_hbm.at[pl.ds(eb, n)],
                              src_b.at[pl.ds(0, n)], sem_i)
        d2 = pltpu.async_copy(dst_hbm.at[pl.ds(eb, n)],
                              dst_b.at[pl.ds(0, n)], sem_i)
        d3 = pltpu.async_copy(val_hbm.at[pl.ds(eb, n)],
                              val_b.at[pl.ds(0, n)], sem_i)
        d1.wait(); d2.wait(); d3.wait()

        def _gl(i, carry):
            b = i * _L
            src_b[pl.ds(b, _L)] = src_b[pl.ds(b, _L)] + t * _N
            return carry
        lax.fori_loop(0, n // _L, _gl, 0)

    def _run_block(nch, first):
        ngrp = nch // _NBG
        for b in range(_NBG):
            _issue_gather(b * _K, gbufs[b], gsems[b])

        def _grp(g, carry):
            qb0 = g * _NBG
            for b in range(_NBG):
                sb = b % 2
                qb = qb0 + b
                _wait_gather(gbufs[b], gsems[b])
                if b < 2:
                    @pl.when(jnp.logical_not(
                        jnp.logical_and(first, g == 0)))
                    def _():
                        _wait_scatter(stgs[sb], dvs[sb], ssems[sb])
                else:
                    _wait_scatter(stgs[sb], dvs[sb], ssems[sb])
                _scale(gbufs[b], stgs[sb], qb * _K)

                @pl.when(g < ngrp - 1)
                def _():
                    _issue_gather((qb + _NBG) * _K, gbufs[b], gsems[b])
                _copy_dst(qb * _K, dvs[sb])
                _issue_scatter(stgs[sb], dvs[sb], ssems[sb])
            return carry
        lax.fori_loop(0, ngrp, _grp, 0)

    for tt in range(_T // _NC):
        t = c * (_T // _NC) + tt

        def _zf(k, carry):
            for j in range(_D // _L):
                stg0[k, pl.ds(j * _L, _L)] = jnp.zeros((_L,), jnp.float32)
            return carry
        lax.fori_loop(0, _K, _zf, 0)
        r0 = s * _RPW

        def _zero_stripe(rows):
            for q in range(rows // _K):
                pltpu.sync_copy(stg0, acc.at[pl.ds(r0 + q * _K, _K)])
            rem = rows - (rows // _K) * _K
            if rem:
                pltpu.sync_copy(stg0.at[pl.ds(0, rem)],
                                acc.at[pl.ds(r0 + (rows // _K) * _K, rem)])

        @pl.when(s < _NS - 1)
        def _():
            _zero_stripe(_RPW)

        @pl.when(s == _NS - 1)
        def _():
            _zero_stripe(_RPW_LAST)

        plsc.subcore_barrier()

        ebase = t * _E + s * _EPW

        def _block(ib, carry):
            _load_idx_block(ebase + ib * _IB, _IB, t)
            _run_block(_CPB, ib == 0)
            return carry
        lax.fori_loop(0, _NIB, _block, 0)

        _wait_scatter(stg0, dst_v0, ss0)
        _wait_scatter(stg1, dst_v1, ss1)

        _load_idx_block(ebase + _NIB * _IB, _TAIL, t)
        off_e = 0
        for j in range(_TAIL // _L):
            dst_tail_v[pl.ds(j * _L, _L)] = dst_b[pl.ds(off_e + j * _L, _L)]
        pltpu.async_copy(xw_hbm.at[src_b.at[pl.ds(off_e, _TAIL)]],
                         g0.at[pl.ds(0, _TAIL)], sg0).wait()
        _scale(g0, stg0, off_e, ngroups=_TAIL // _L)
        pltpu.sync_copy(stg0.at[pl.ds(0, _TAIL)], acc.at[dst_tail_v],
                        add=True)

        plsc.subcore_barrier()

        obase = t * _N + r0

        @pl.when(s < _NS - 1)
        def _():
            pltpu.sync_copy(acc.at[pl.ds(r0, _RPW)],
                            out_hbm.at[pl.ds(obase, _RPW)])

        @pl.when(s == _NS - 1)
        def _():
            pltpu.sync_copy(acc.at[pl.ds(r0, _RPW_LAST)],
                            out_hbm.at[pl.ds(obase, _RPW_LAST)])


def kernel(adj_indices, adj_values, input, M, W):
    dst = adj_indices[:, 0, :].reshape(-1)
    src = adj_indices[:, 1, :].reshape(-1)
    val = adj_values.reshape(-1)
    x_flat = input.reshape(_T * _N, _D)
    xw = _xw_matmul(x_flat, W)

    mesh = plsc.VectorSubcoreMesh(core_axis_name="c", subcore_axis_name="s")
    spmm = pl.kernel(
        _spmm_body,
        out_type=jax.ShapeDtypeStruct((_T * _N, _D), jnp.float32),
        compiler_params=pltpu.CompilerParams(use_tc_tiling_on_sc=False),
        mesh=mesh,
        scratch_types=[
            pltpu.VMEM_SHARED((_N, _D), jnp.float32),
            pltpu.VMEM((_IB,), jnp.int32),
            pltpu.VMEM((_IB,), jnp.int32),
            pltpu.VMEM((_IB,), jnp.float32),
            pltpu.VMEM((_K,), jnp.int32),
            pltpu.VMEM((_K,), jnp.int32),
            pltpu.VMEM((_TAIL,), jnp.int32),
            pltpu.VMEM((_K, _D), jnp.float32),
            pltpu.VMEM((_K, _D), jnp.float32),
            pltpu.VMEM((_K, _D), jnp.float32),
            pltpu.VMEM((_K, _D), jnp.float32),
            pltpu.VMEM((_K, _D), jnp.float32),
            pltpu.VMEM((_K, _D), jnp.float32),
            pltpu.SemaphoreType.DMA,
            pltpu.SemaphoreType.DMA,
            pltpu.SemaphoreType.DMA,
            pltpu.SemaphoreType.DMA,
            pltpu.SemaphoreType.DMA,
            pltpu.SemaphoreType.DMA,
            pltpu.SemaphoreType.DMA,
        ],
    )
    out = spmm(dst, src, val, xw)
    return out.reshape(_T, _N, _D)
